# initial kernel scaffold (unmeasured)
import jax
import jax.numpy as jnp
from jax import lax
from jax.experimental import pallas as pl
from jax.experimental.pallas import tpu as pltpu

NDEV = 16


def kernel(x, w_mat):
    m_tot, k_sh = x.shape
    k_tot, n_tot = w_mat.shape
    m_sh = m_tot // NDEV

    def a2a_body(x_ref, out_ref, send_buf, comm, send_sems, recv_sems):
        me = lax.axis_index("i")
        out_ref[:, pl.ds(me * k_sh, k_sh)] = x_ref[
            pl.ds(me * m_sh, m_sh), :
        ].astype(jnp.bfloat16)
        rdmas = []
        for d in range(1, NDEV):
            tgt = lax.rem(me + d, NDEV)
            send_buf[d, :, :] = x_ref[pl.ds(tgt * m_sh, m_sh), :].astype(
                jnp.bfloat16
            )
            rdma = pltpu.make_async_remote_copy(
                src_ref=send_buf.at[d],
                dst_ref=comm.at[d],
                send_sem=send_sems.at[d],
                recv_sem=recv_sems.at[d],
                device_id=(tgt,),
                device_id_type=pl.DeviceIdType.MESH,
            )
            rdma.start()
            rdmas.append(rdma)
        for d in range(1, NDEV):
            rdmas[d - 1].wait_recv()
            src = lax.rem(me - d + NDEV, NDEV)
            out_ref[:, pl.ds(src * k_sh, k_sh)] = comm[d, :, :]
        for r in rdmas:
            r.wait_send()

    x_rows = pl.pallas_call(
        a2a_body,
        out_shape=jax.ShapeDtypeStruct((m_sh, k_tot), jnp.bfloat16),
        in_specs=[pl.BlockSpec(memory_space=pltpu.VMEM)],
        out_specs=pl.BlockSpec(memory_space=pltpu.VMEM),
        scratch_shapes=[
            pltpu.VMEM((NDEV, m_sh, k_sh), jnp.bfloat16),
            pltpu.VMEM((NDEV, m_sh, k_sh), jnp.bfloat16),
            pltpu.SemaphoreType.DMA((NDEV,)),
            pltpu.SemaphoreType.DMA((NDEV,)),
        ],
        compiler_params=pltpu.CompilerParams(collective_id=0),
    )(x)

    nb = 512
    n_blocks = n_tot // nb

    def gemm_body(x_ref, w_ref, y_ref, amax_ref, acc):
        j = pl.program_id(0)
        w_bf = w_ref[...].astype(jnp.bfloat16)
        y = jnp.dot(x_ref[...], w_bf, preferred_element_type=jnp.float32)
        y_ref[...] = y
        m = jnp.max(jnp.abs(y))

        @pl.when(j == 0)
        def _():
            acc[0] = m

        @pl.when(j > 0)
        def _():
            acc[0] = jnp.maximum(acc[0], m)

        amax_ref[0, 0] = acc[0]

    y, amax_local = pl.pallas_call(
        gemm_body,
        grid=(n_blocks,),
        in_specs=[
            pl.BlockSpec((m_sh, k_tot), lambda j: (0, 0), memory_space=pltpu.VMEM),
            pl.BlockSpec((k_tot, nb), lambda j: (0, j), memory_space=pltpu.VMEM),
        ],
        out_specs=[
            pl.BlockSpec((m_sh, nb), lambda j: (0, j), memory_space=pltpu.VMEM),
            pl.BlockSpec((1, 1), lambda j: (0, 0), memory_space=pltpu.SMEM),
        ],
        out_shape=[
            jax.ShapeDtypeStruct((m_sh, n_tot), jnp.float32),
            jax.ShapeDtypeStruct((1, 1), jnp.float32),
        ],
        scratch_shapes=[pltpu.SMEM((1,), jnp.float32)],
    )(x_rows, w_mat)

    def allmax_body(a_ref, out_ref, bcast, comm, send_sems, recv_sems):
        me = lax.axis_index("i")
        bcast[...] = jnp.full((8, 128), a_ref[0, 0], jnp.float32)
        rdmas = []
        for d in range(1, NDEV):
            tgt = lax.rem(me + d, NDEV)
            rdma = pltpu.make_async_remote_copy(
                src_ref=bcast,
                dst_ref=comm.at[d],
                send_sem=send_sems.at[d],
                recv_sem=recv_sems.at[d],
                device_id=(tgt,),
                device_id_type=pl.DeviceIdType.MESH,
            )
            rdma.start()
            rdmas.append(rdma)
        m = a_ref[0, 0]
        for d in range(1, NDEV):
            rdmas[d - 1].wait_recv()
            m = jnp.maximum(m, jnp.max(comm[d, :, :]))
        out_ref[0, 0] = m
        for r in rdmas:
            r.wait_send()

    amax = pl.pallas_call(
        allmax_body,
        out_shape=jax.ShapeDtypeStruct((1, 1), jnp.float32),
        in_specs=[pl.BlockSpec(memory_space=pltpu.SMEM)],
        out_specs=pl.BlockSpec(memory_space=pltpu.SMEM),
        scratch_shapes=[
            pltpu.VMEM((8, 128), jnp.float32),
            pltpu.VMEM((NDEV, 8, 128), jnp.float32),
            pltpu.SemaphoreType.DMA((NDEV,)),
            pltpu.SemaphoreType.DMA((NDEV,)),
        ],
        compiler_params=pltpu.CompilerParams(collective_id=1),
    )(amax_local)

    def quant_body(a_ref, y_ref, out_ref):
        scale = a_ref[0, 0] / 448.0
        q = (y_ref[...] / scale).astype(jnp.float8_e4m3fn)
        out_ref[...] = q.astype(jnp.float32) * scale

    out = pl.pallas_call(
        quant_body,
        out_shape=jax.ShapeDtypeStruct((m_sh, n_tot), jnp.float32),
        in_specs=[
            pl.BlockSpec(memory_space=pltpu.SMEM),
            pl.BlockSpec(memory_space=pltpu.VMEM),
        ],
        out_specs=pl.BlockSpec(memory_space=pltpu.VMEM),
    )(amax, y)

    return out


# baseline (device time: 101715 ns/iter reference)
import jax
import jax.numpy as jnp
from jax import lax
from jax.experimental import pallas as pl
from jax.experimental.pallas import tpu as pltpu

NDEV = 16


def kernel(x, w_mat):
    m_tot, k_sh = x.shape
    k_tot, n_tot = w_mat.shape
    m_sh = m_tot // NDEV

    def a2a_body(x_ref, out_ref, send_buf, comm, send_sems, recv_sems):
        me = lax.axis_index("i")
        out_ref[:, pl.ds(me * k_sh, k_sh)] = x_ref[
            pl.ds(me * m_sh, m_sh), :
        ].astype(jnp.bfloat16)
        rdmas = []
        for d in range(1, NDEV):
            tgt = lax.rem(me + d, NDEV)
            send_buf[d, :, :] = x_ref[pl.ds(tgt * m_sh, m_sh), :].astype(
                jnp.bfloat16
            )
            rdma = pltpu.make_async_remote_copy(
                src_ref=send_buf.at[d],
                dst_ref=comm.at[d],
                send_sem=send_sems.at[d],
                recv_sem=recv_sems.at[d],
                device_id=(tgt,),
                device_id_type=pl.DeviceIdType.MESH,
            )
            rdma.start()
            rdmas.append(rdma)
        for d in range(1, NDEV):
            rdmas[d - 1].wait_recv()
            src = lax.rem(me - d + NDEV, NDEV)
            out_ref[:, pl.ds(src * k_sh, k_sh)] = comm[d, :, :]
        for r in rdmas:
            r.wait_send()

    x_rows = pl.pallas_call(
        a2a_body,
        out_shape=jax.ShapeDtypeStruct((m_sh, k_tot), jnp.bfloat16),
        in_specs=[pl.BlockSpec(memory_space=pltpu.VMEM)],
        out_specs=pl.BlockSpec(memory_space=pltpu.VMEM),
        scratch_shapes=[
            pltpu.VMEM((NDEV, m_sh, k_sh), jnp.bfloat16),
            pltpu.VMEM((NDEV, m_sh, k_sh), jnp.bfloat16),
            pltpu.SemaphoreType.DMA((NDEV,)),
            pltpu.SemaphoreType.DMA((NDEV,)),
        ],
    )(x)

    nb = 512
    n_blocks = n_tot // nb

    def gemm_body(x_ref, w_ref, y_ref, amax_ref, acc):
        j = pl.program_id(0)
        w_bf = w_ref[...].astype(jnp.bfloat16)
        y = jnp.dot(x_ref[...], w_bf, preferred_element_type=jnp.float32)
        y_ref[...] = y
        m = jnp.max(jnp.abs(y))

        @pl.when(j == 0)
        def _():
            acc[0] = m

        @pl.when(j > 0)
        def _():
            acc[0] = jnp.maximum(acc[0], m)

        amax_ref[0, 0] = acc[0]

    y, amax_local = pl.pallas_call(
        gemm_body,
        grid=(n_blocks,),
        in_specs=[
            pl.BlockSpec((m_sh, k_tot), lambda j: (0, 0), memory_space=pltpu.VMEM),
            pl.BlockSpec((k_tot, nb), lambda j: (0, j), memory_space=pltpu.VMEM),
        ],
        out_specs=[
            pl.BlockSpec((m_sh, nb), lambda j: (0, j), memory_space=pltpu.VMEM),
            pl.BlockSpec((1, 1), lambda j: (0, 0), memory_space=pltpu.SMEM),
        ],
        out_shape=[
            jax.ShapeDtypeStruct((m_sh, n_tot), jnp.float32),
            jax.ShapeDtypeStruct((1, 1), jnp.float32),
        ],
        scratch_shapes=[pltpu.SMEM((1,), jnp.float32)],
    )(x_rows, w_mat)

    def allmax_body(a_ref, out_ref, bcast, comm, send_sems, recv_sems):
        me = lax.axis_index("i")
        bcast[...] = jnp.full((8, 128), a_ref[0, 0], jnp.float32)
        rdmas = []
        for d in range(1, NDEV):
            tgt = lax.rem(me + d, NDEV)
            rdma = pltpu.make_async_remote_copy(
                src_ref=bcast,
                dst_ref=comm.at[d],
                send_sem=send_sems.at[d],
                recv_sem=recv_sems.at[d],
                device_id=(tgt,),
                device_id_type=pl.DeviceIdType.MESH,
            )
            rdma.start()
            rdmas.append(rdma)
        m = a_ref[0, 0]
        for d in range(1, NDEV):
            rdmas[d - 1].wait_recv()
            m = jnp.maximum(m, jnp.max(comm[d, :, :]))
        out_ref[0, 0] = m
        for r in rdmas:
            r.wait_send()

    amax = pl.pallas_call(
        allmax_body,
        out_shape=jax.ShapeDtypeStruct((1, 1), jnp.float32),
        in_specs=[pl.BlockSpec(memory_space=pltpu.SMEM)],
        out_specs=pl.BlockSpec(memory_space=pltpu.SMEM),
        scratch_shapes=[
            pltpu.VMEM((8, 128), jnp.float32),
            pltpu.VMEM((NDEV, 8, 128), jnp.float32),
            pltpu.SemaphoreType.DMA((NDEV,)),
            pltpu.SemaphoreType.DMA((NDEV,)),
        ],
    )(amax_local)

    def quant_body(a_ref, y_ref, out_ref):
        scale = a_ref[0, 0] / 448.0
        q = (y_ref[...] / scale).astype(jnp.float8_e4m3fn)
        out_ref[...] = q.astype(jnp.float32) * scale

    out = pl.pallas_call(
        quant_body,
        out_shape=jax.ShapeDtypeStruct((m_sh, n_tot), jnp.float32),
        in_specs=[
            pl.BlockSpec(memory_space=pltpu.SMEM),
            pl.BlockSpec(memory_space=pltpu.VMEM),
        ],
        out_specs=pl.BlockSpec(memory_space=pltpu.VMEM),
    )(amax, y)

    return out


# device time: 101661 ns/iter; 1.0005x vs baseline; 1.0005x over previous
import jax
import jax.numpy as jnp
from jax import lax
from jax.experimental import pallas as pl
from jax.experimental.pallas import tpu as pltpu

NDEV = 16


def kernel(x, w_mat):
    m_tot, k_sh = x.shape
    k_tot, n_tot = w_mat.shape
    m_sh = m_tot // NDEV
    NC = 2048
    NCHUNKS = n_tot // NC

    def body(
        x_ref,
        w_ref,
        out_ref,
        send_buf,
        comm,
        wbuf,
        y_acc,
        amax_bcast,
        amax_comm,
        send_sems,
        recv_sems,
        amax_send_sems,
        amax_recv_sems,
        wdma_sems,
    ):
        me = lax.axis_index("i")

        rdmas = []
        for d in range(1, NDEV):
            tgt = lax.rem(me + d, NDEV)
            send_buf[d, :, :] = x_ref[pl.ds(tgt * m_sh, m_sh), :].astype(
                jnp.bfloat16
            )
            rdma = pltpu.make_async_remote_copy(
                src_ref=send_buf.at[d],
                dst_ref=comm.at[d],
                send_sem=send_sems.at[d],
                recv_sem=recv_sems.at[d],
                device_id=(tgt,),
                device_id_type=pl.DeviceIdType.MESH,
            )
            rdma.start()
            rdmas.append(rdma)

        comm[0, :, :] = x_ref[pl.ds(me * m_sh, m_sh), :].astype(jnp.bfloat16)

        def w_dma(t, c, slot):
            src = lax.rem(me - t + NDEV, NDEV)
            return pltpu.make_async_copy(
                w_ref.at[pl.ds(src * m_sh, m_sh), pl.ds(c * NC, NC)],
                wbuf.at[slot],
                wdma_sems.at[slot],
            )

        def recv_desc(t):
            return pltpu.make_async_remote_copy(
                src_ref=comm.at[t],
                dst_ref=comm.at[t],
                send_sem=send_sems.at[0],
                recv_sem=recv_sems.at[t],
                device_id=(0,),
                device_id_type=pl.DeviceIdType.MESH,
            )

        total = NDEV * NCHUNKS
        w_dma(0, 0, 0).start()

        def step(idx, _):
            t = idx // NCHUNKS
            c = lax.rem(idx, NCHUNKS)

            @pl.when(idx + 1 < total)
            def _():
                nxt = idx + 1
                w_dma(nxt // NCHUNKS, lax.rem(nxt, NCHUNKS), lax.rem(nxt, 2)).start()

            w_dma(t, c, lax.rem(idx, 2)).wait()

            @pl.when(jnp.logical_and(c == 0, t > 0))
            def _():
                recv_desc(t).wait_recv()

            xb = comm[t, :, :]
            w_bf = wbuf[lax.rem(idx, 2)].astype(jnp.bfloat16)
            contrib = jnp.dot(xb, w_bf, preferred_element_type=jnp.float32)
            sl = pl.ds(c * NC, NC)

            @pl.when(t == 0)
            def _():
                y_acc[:, sl] = contrib

            @pl.when(t > 0)
            def _():
                y_acc[:, sl] += contrib

            return None

        lax.fori_loop(0, total, step, None, unroll=False)

        m = jnp.max(jnp.abs(y_acc[:, pl.ds(0, NC)]))
        for c in range(1, NCHUNKS):
            m = jnp.maximum(m, jnp.max(jnp.abs(y_acc[:, pl.ds(c * NC, NC)])))
        amax_bcast[...] = jnp.full((8, 128), m, jnp.float32)
        amax_rdmas = []
        for d in range(1, NDEV):
            tgt = lax.rem(me + d, NDEV)
            rdma = pltpu.make_async_remote_copy(
                src_ref=amax_bcast,
                dst_ref=amax_comm.at[d],
                send_sem=amax_send_sems.at[d],
                recv_sem=amax_recv_sems.at[d],
                device_id=(tgt,),
                device_id_type=pl.DeviceIdType.MESH,
            )
            rdma.start()
            amax_rdmas.append(rdma)
        for d in range(1, NDEV):
            amax_rdmas[d - 1].wait_recv()
            m = jnp.maximum(m, jnp.max(amax_comm[d, :, :]))

        for r in rdmas:
            r.wait_send()
        for r in amax_rdmas:
            r.wait_send()

        scale = m / 448.0
        inv = 448.0 / m
        for c in range(NCHUNKS):
            sl = pl.ds(c * NC, NC)
            q = (y_acc[:, sl] * inv).astype(jnp.float8_e4m3fn)
            out_ref[:, sl] = q.astype(jnp.float32) * scale

    return pl.pallas_call(
        body,
        out_shape=jax.ShapeDtypeStruct((m_sh, n_tot), jnp.float32),
        in_specs=[
            pl.BlockSpec(memory_space=pltpu.VMEM),
            pl.BlockSpec(memory_space=pl.ANY),
        ],
        out_specs=pl.BlockSpec(memory_space=pltpu.VMEM),
        scratch_shapes=[
            pltpu.VMEM((NDEV, m_sh, k_sh), jnp.bfloat16),
            pltpu.VMEM((NDEV, m_sh, k_sh), jnp.bfloat16),
            pltpu.VMEM((2, m_sh, NC), jnp.float32),
            pltpu.VMEM((m_sh, n_tot), jnp.float32),
            pltpu.VMEM((8, 128), jnp.float32),
            pltpu.VMEM((NDEV, 8, 128), jnp.float32),
            pltpu.SemaphoreType.DMA((NDEV,)),
            pltpu.SemaphoreType.DMA((NDEV,)),
            pltpu.SemaphoreType.DMA((NDEV,)),
            pltpu.SemaphoreType.DMA((NDEV,)),
            pltpu.SemaphoreType.DMA((2,)),
        ],
    )(x, w_mat)


# device time: 76341 ns/iter; 1.3324x vs baseline; 1.3317x over previous
import jax
import jax.numpy as jnp
from jax import lax
from jax.experimental import pallas as pl
from jax.experimental.pallas import tpu as pltpu

NDEV = 16


def kernel(x, w_mat):
    m_tot, k_sh = x.shape
    k_tot, n_tot = w_mat.shape
    m_sh = m_tot // NDEV
    NC = 2048
    NCHUNKS = n_tot // NC
    NG = 4
    GK = k_tot // NG
    SRC_PER_G = NDEV // NG
    NSLOT = 2

    def body(
        x_ref,
        w_ref,
        out_ref,
        send_buf,
        x_rows,
        wbuf,
        y_acc,
        amax_bcast,
        amax_comm,
        send_sems,
        recv_sems,
        amax_send_sems,
        amax_recv_sems,
        wdma_sems,
    ):
        me = lax.axis_index("i")

        my_col = pl.ds(me * k_sh, k_sh)
        rdmas = []
        for d in range(1, NDEV):
            tgt = lax.rem(me + d, NDEV)
            send_buf[d, :, :] = x_ref[pl.ds(tgt * m_sh, m_sh), :].astype(
                jnp.bfloat16
            )
            rdma = pltpu.make_async_remote_copy(
                src_ref=send_buf.at[d],
                dst_ref=x_rows.at[:, my_col],
                send_sem=send_sems.at[d],
                recv_sem=recv_sems.at[d],
                device_id=(tgt,),
                device_id_type=pl.DeviceIdType.MESH,
            )
            rdma.start()
            rdmas.append(rdma)
        x_rows[:, my_col] = x_ref[pl.ds(me * m_sh, m_sh), :].astype(jnp.bfloat16)

        g_first = me // SRC_PER_G

        def grp_of(j):
            return lax.rem(g_first + j, NG)

        def start_wdma(idx):
            j, c, slot = idx // NCHUNKS, idx % NCHUNKS, idx % NSLOT
            grp = grp_of(j)
            for g in range(NG):

                @pl.when(grp == g)
                def _(g=g):
                    pltpu.make_async_copy(
                        w_ref.at[pl.ds(g * GK, GK), pl.ds(c * NC, NC)],
                        wbuf.at[slot],
                        wdma_sems.at[slot],
                    ).start()

        def wait_wdma(idx):
            slot = idx % NSLOT
            c = idx % NCHUNKS
            pltpu.make_async_copy(
                w_ref.at[pl.ds(0, GK), pl.ds(c * NC, NC)],
                wbuf.at[slot],
                wdma_sems.at[slot],
            ).wait()

        def recv_wait(s):
            d = lax.rem(me - s + NDEV, NDEV)
            pltpu.make_async_remote_copy(
                src_ref=send_buf.at[0],
                dst_ref=x_rows.at[:, pl.ds(s * k_sh, k_sh)],
                send_sem=send_sems.at[0],
                recv_sem=recv_sems.at[d],
                device_id=(0,),
                device_id_type=pl.DeviceIdType.MESH,
            ).wait_recv()

        total = NG * NCHUNKS
        for p in range(NSLOT - 1):
            start_wdma(p)

        for idx in range(total):
            j, c = idx // NCHUNKS, idx % NCHUNKS
            if idx + NSLOT - 1 < total:
                start_wdma(idx + NSLOT - 1)
            if c == 0:
                grp = grp_of(j)
                for si in range(SRC_PER_G):
                    s = grp * SRC_PER_G + si

                    @pl.when(s != me)
                    def _(s=s):
                        recv_wait(s)

            wait_wdma(idx)
            xg = x_rows[:, pl.ds(grp_of(j) * GK, GK)]
            contrib = jnp.dot(
                xg,
                wbuf[idx % NSLOT],
                preferred_element_type=jnp.float32,
                precision=lax.Precision.DEFAULT,
            )
            sl = pl.ds(c * NC, NC)
            if j == 0:
                y_acc[:, sl] = contrib
            else:
                y_acc[:, sl] += contrib

        m = jnp.max(jnp.abs(y_acc[:, pl.ds(0, NC)]))
        for c in range(1, NCHUNKS):
            m = jnp.maximum(m, jnp.max(jnp.abs(y_acc[:, pl.ds(c * NC, NC)])))
        amax_bcast[...] = jnp.full((8, 128), m, jnp.float32)
        amax_rdmas = []
        for d in range(1, NDEV):
            tgt = lax.rem(me + d, NDEV)
            rdma = pltpu.make_async_remote_copy(
                src_ref=amax_bcast,
                dst_ref=amax_comm.at[d],
                send_sem=amax_send_sems.at[d],
                recv_sem=amax_recv_sems.at[d],
                device_id=(tgt,),
                device_id_type=pl.DeviceIdType.MESH,
            )
            rdma.start()
            amax_rdmas.append(rdma)
        for d in range(1, NDEV):
            amax_rdmas[d - 1].wait_recv()
            m = jnp.maximum(m, jnp.max(amax_comm[d, :, :]))

        for r in rdmas:
            r.wait_send()
        for r in amax_rdmas:
            r.wait_send()

        scale = m / 448.0
        inv = 448.0 / m
        for c in range(NCHUNKS):
            sl = pl.ds(c * NC, NC)
            q = (y_acc[:, sl] * inv).astype(jnp.float8_e4m3fn)
            out_ref[:, sl] = q.astype(jnp.float32) * scale

    return pl.pallas_call(
        body,
        out_shape=jax.ShapeDtypeStruct((m_sh, n_tot), jnp.float32),
        in_specs=[
            pl.BlockSpec(memory_space=pltpu.VMEM),
            pl.BlockSpec(memory_space=pl.ANY),
        ],
        out_specs=pl.BlockSpec(memory_space=pltpu.VMEM),
        scratch_shapes=[
            pltpu.VMEM((NDEV, m_sh, k_sh), jnp.bfloat16),
            pltpu.VMEM((m_sh, k_tot), jnp.bfloat16),
            pltpu.VMEM((NSLOT, GK, NC), jnp.float32),
            pltpu.VMEM((m_sh, n_tot), jnp.float32),
            pltpu.VMEM((8, 128), jnp.float32),
            pltpu.VMEM((NDEV, 8, 128), jnp.float32),
            pltpu.SemaphoreType.DMA((NDEV,)),
            pltpu.SemaphoreType.DMA((NDEV,)),
            pltpu.SemaphoreType.DMA((NDEV,)),
            pltpu.SemaphoreType.DMA((NDEV,)),
            pltpu.SemaphoreType.DMA((NSLOT,)),
        ],
    )(x, w_mat)
